# baseline (speedup 1.0000x reference)
import functools

import jax
import jax.numpy as jnp
from jax import lax
from jax.experimental import pallas as pl
from jax.experimental.pallas import tpu as pltpu
from jax.experimental.pallas import tpu_sc as plsc

_P = 20
_K = 128
_CH = 32
_NBUF = 3
_W2 = 2 * (_P + _K)

_PLAN = (
    (0, 24, 32), (1, 56, 32), (2, 88, 32), (3, 120, 24),
    (4, 168, 32), (5, 200, 32), (6, 232, 32), (7, 264, 32),
)


def _embed_call(idx_chunks, f16_idx, pA, pB, pcv, embed_table, B, D, new_len):
    info = plsc.get_sparse_core_info()
    NC, NS = info.num_cores, info.num_subcores
    NW = NC * NS

    mesh = plsc.VectorSubcoreMesh(core_axis_name="c", subcore_axis_name="s")

    @functools.partial(
        pl.kernel,
        out_type=jax.ShapeDtypeStruct((B * new_len, D), jnp.float32),
        mesh=mesh,
        scratch_types=[
            pltpu.VMEM((8, _CH), jnp.int32),
            pltpu.VMEM((16,), jnp.int32),
            pltpu.VMEM((8, D), jnp.float32),
            pltpu.VMEM((_NBUF, _CH, D), jnp.float32),
            pltpu.SemaphoreType.DMA((_NBUF,)),
            pltpu.SemaphoreType.DMA((_NBUF,)),
            pltpu.SemaphoreType.DMA,
        ],
        compiler_params=pltpu.CompilerParams(use_tc_tiling_on_sc=True),
    )
    def sc_embed(idx_hbm, f16_hbm, pa_hbm, pb_hbm, pcv_hbm, table_hbm,
                 out_hbm, idx_v, f16_v, pcv_v, rows_v, gsem, osem, psem):
        wid = lax.axis_index("s") * NC + lax.axis_index("c")
        base = wid * _W2
        pltpu.sync_copy(idx_hbm.at[wid], idx_v)
        pltpu.sync_copy(f16_hbm.at[wid], f16_v)
        pltpu.sync_copy(pcv_hbm, pcv_v)

        pa_d = pltpu.async_copy(pa_hbm, out_hbm.at[pl.ds(base, 16), :], psem)
        pb_d = pltpu.async_copy(pb_hbm, out_hbm.at[pl.ds(base + 152, 16), :],
                                psem)

        def start_g(j):
            slot = j % _NBUF
            if j == 0:
                return pltpu.async_copy(
                    table_hbm.at[f16_v], rows_v.at[0, pl.ds(0, 16)],
                    gsem.at[slot])
            r, _, _ = _PLAN[j - 1]
            return pltpu.async_copy(
                table_hbm.at[idx_v.at[r]], rows_v.at[slot], gsem.at[slot])

        def start_o(j):
            slot = j % _NBUF
            if j == 0:
                for r_dst, r_src in ((0, 0), (1, 1), (2, 2), (3, 3),
                                     (12, 4), (13, 5), (14, 6), (15, 7)):
                    for c in range(D // 16):
                        rows_v[0, r_dst, pl.ds(c * 16, 16)] = (
                            pcv_v[r_src, pl.ds(c * 16, 16)])
                return [
                    pltpu.async_copy(
                        rows_v.at[0, pl.ds(0, 8)],
                        out_hbm.at[pl.ds(base + 16, 8), :], osem.at[slot]),
                    pltpu.async_copy(
                        rows_v.at[0, pl.ds(8, 8)],
                        out_hbm.at[pl.ds(base + 144, 8), :], osem.at[slot]),
                ]
            _, off, rows = _PLAN[j - 1]
            return [pltpu.async_copy(
                rows_v.at[slot, pl.ds(0, rows)],
                out_hbm.at[pl.ds(base + off, rows), :], osem.at[slot])]

        n_jobs = 1 + len(_PLAN)
        gd = {j: start_g(j) for j in range(_NBUF)}
        od = {}
        for j in range(n_jobs):
            gd[j].wait()
            od[j] = start_o(j)
            if j + _NBUF < n_jobs:
                for d in od[j]:
                    d.wait()
                gd[j + _NBUF] = start_g(j + _NBUF)
        for j in range(max(0, n_jobs - _NBUF), n_jobs):
            for d in od[j]:
                d.wait()
        pa_d.wait()
        pb_d.wait()

    return sc_embed(idx_chunks, f16_idx, pA, pB, pcv, embed_table)


def _mask_labels_call(am3, lab3, B, NB):
    def body(am_ref, lab_ref, mask_ref, labout_ref):
        mask_ref[...] = jnp.concatenate(
            [jnp.ones((B, NB, 1, _P), jnp.int32), am_ref[...]], axis=3)
        labout_ref[...] = jnp.concatenate(
            [jnp.full((B, NB, 1, _P), -100, jnp.int32), lab_ref[...]], axis=3)

    out_sd = jax.ShapeDtypeStruct((B, NB, 1, _P + _K), jnp.int32)
    return pl.pallas_call(body, out_shape=(out_sd, out_sd))(am3, lab3)


def kernel(input_ids, attention_mask, labels, embed_table, prompt_embed):
    B, T = input_ids.shape
    V, D = embed_table.shape
    NB = T // _K
    new_len = NB * (_K + _P)
    NW = 32

    ids2 = input_ids.reshape(NW, 2, _K)
    even, odd = ids2[:, 0], ids2[:, 1]
    ec3 = jnp.stack(
        [even[:, 4:36], even[:, 36:68], even[:, 68:100],
         jnp.concatenate([even[:, 100:124], even[:, 120:128]], axis=1)],
        axis=1)
    oc3 = odd.reshape(NW, 4, _CH)
    idx_chunks = jnp.concatenate([ec3, oc3], axis=1)
    dummy = jnp.broadcast_to(even[:, 0:1], (NW, 4))
    f16_idx = jnp.concatenate(
        [dummy, even[:, 0:4], even[:, 124:128], dummy], axis=1)

    pA = prompt_embed[0:16]
    pB = prompt_embed[4:20]
    pcv = jnp.concatenate([prompt_embed[16:20], prompt_embed[0:4]], axis=0)

    out2 = _embed_call(idx_chunks, f16_idx, pA, pB, pcv, embed_table,
                       B, D, new_len)
    out = out2.reshape(B, new_len, D)

    am3 = attention_mask.reshape(B, NB, 1, _K)
    lab3 = labels.reshape(B, NB, 1, _K)
    mask3, lab_out3 = _mask_labels_call(am3, lab3, B, NB)
    return out, mask3.reshape(B, new_len), lab_out3.reshape(B, new_len)


# speedup vs baseline: 2.8154x; 2.8154x over previous
import functools

import jax
import jax.numpy as jnp
from jax import lax
from jax.experimental import pallas as pl
from jax.experimental.pallas import tpu as pltpu
from jax.experimental.pallas import tpu_sc as plsc

_P = 20
_K = 128
_CH = 32
_NBUF = 2
_W2 = 2 * (_P + _K)

_PLAN = (
    (0, 24, 32), (1, 56, 32), (2, 88, 32), (3, 120, 24),
    (4, 168, 32), (5, 200, 32), (6, 232, 32), (7, 264, 32),
)


def _embed_call(idx_chunks, f16_idx, pv32, embed_table, B, D, new_len):
    info = plsc.get_sparse_core_info()
    NC, NS = info.num_cores, info.num_subcores
    NW = NC * NS

    mesh = plsc.VectorSubcoreMesh(core_axis_name="c", subcore_axis_name="s")

    @functools.partial(
        pl.kernel,
        out_type=jax.ShapeDtypeStruct((B * new_len, D), jnp.float32),
        mesh=mesh,
        scratch_types=[
            pltpu.VMEM((8, _CH), jnp.int32),
            pltpu.VMEM((16,), jnp.int32),
            pltpu.VMEM((32, D), jnp.float32),
            pltpu.VMEM((16, D), jnp.float32),
            pltpu.VMEM((_NBUF, _CH, D), jnp.float32),
            pltpu.SemaphoreType.DMA((_NBUF,)),
            pltpu.SemaphoreType.DMA((_NBUF,)),
            pltpu.SemaphoreType.DMA((3,)),
            pltpu.SemaphoreType.DMA,
            pltpu.SemaphoreType.DMA,
        ],
        compiler_params=pltpu.CompilerParams(use_tc_tiling_on_sc=True),
    )
    def sc_embed(idx_hbm, f16_hbm, pv_hbm, table_hbm, out_hbm,
                 idx_v, f16_v, pv_v, fm_v, rows_v,
                 gsem, osem, lsem, fsem, psem):
        wid = lax.axis_index("s") * NC + lax.axis_index("c")
        base = wid * _W2
        l1 = pltpu.async_copy(idx_hbm.at[wid], idx_v, lsem.at[0])
        l2 = pltpu.async_copy(f16_hbm.at[wid], f16_v, lsem.at[1])
        l3 = pltpu.async_copy(pv_hbm, pv_v, lsem.at[2])
        l1.wait()
        l2.wait()

        fg = pltpu.async_copy(table_hbm.at[f16_v], fm_v, fsem)

        def start_g(j):
            r, _, _ = _PLAN[j]
            return pltpu.async_copy(
                table_hbm.at[idx_v.at[r]], rows_v.at[j % _NBUF],
                gsem.at[j % _NBUF])

        def start_o(j):
            _, off, rows = _PLAN[j]
            return pltpu.async_copy(
                rows_v.at[j % _NBUF, pl.ds(0, rows)],
                out_hbm.at[pl.ds(base + off, rows), :], osem.at[j % _NBUF])

        n_jobs = len(_PLAN)
        gd = {j: start_g(j) for j in range(_NBUF)}
        od = {}
        pd = []
        for j in range(n_jobs):
            gd[j].wait()
            od[j] = start_o(j)
            if j == 0:
                l3.wait()
                pd.append(pltpu.async_copy(
                    pv_v.at[pl.ds(0, 16)],
                    out_hbm.at[pl.ds(base, 16), :], psem))
                pd.append(pltpu.async_copy(
                    pv_v.at[pl.ds(16, 16)],
                    out_hbm.at[pl.ds(base + 152, 16), :], psem))
            if j == 1:
                fg.wait()
                for r_dst, r_src in ((0, 28), (1, 29), (2, 30), (3, 31),
                                     (12, 0), (13, 1), (14, 2), (15, 3)):
                    for c in range(D // 16):
                        fm_v[r_dst, pl.ds(c * 16, 16)] = (
                            pv_v[r_src, pl.ds(c * 16, 16)])
                pd.append(pltpu.async_copy(
                    fm_v.at[pl.ds(0, 8)],
                    out_hbm.at[pl.ds(base + 16, 8), :], psem))
                pd.append(pltpu.async_copy(
                    fm_v.at[pl.ds(8, 8)],
                    out_hbm.at[pl.ds(base + 144, 8), :], psem))
            if j + _NBUF < n_jobs:
                od[j].wait()
                gd[j + _NBUF] = start_g(j + _NBUF)
        for j in range(max(0, n_jobs - _NBUF), n_jobs):
            od[j].wait()
        for d in pd:
            d.wait()

    return sc_embed(idx_chunks, f16_idx, pv32, embed_table)


def _mask_labels_call(am3, lab3, B, NB):
    def body(am_ref, lab_ref, mask_ref, labout_ref):
        mask_ref[...] = jnp.concatenate(
            [jnp.ones((B, NB, 1, _P), jnp.int32), am_ref[...]], axis=3)
        labout_ref[...] = jnp.concatenate(
            [jnp.full((B, NB, 1, _P), -100, jnp.int32), lab_ref[...]], axis=3)

    out_sd = jax.ShapeDtypeStruct((B, NB, 1, _P + _K), jnp.int32)
    return pl.pallas_call(body, out_shape=(out_sd, out_sd))(am3, lab3)


def kernel(input_ids, attention_mask, labels, embed_table, prompt_embed):
    B, T = input_ids.shape
    V, D = embed_table.shape
    NB = T // _K
    new_len = NB * (_K + _P)
    NW = 32

    ids2 = input_ids.reshape(NW, 2, _K)
    even, odd = ids2[:, 0], ids2[:, 1]
    ec3 = jnp.stack(
        [even[:, 4:36], even[:, 36:68], even[:, 68:100],
         jnp.concatenate([even[:, 100:124], even[:, 120:128]], axis=1)],
        axis=1)
    oc3 = odd.reshape(NW, 4, _CH)
    idx_chunks = jnp.concatenate([ec3, oc3], axis=1)
    dummy = jnp.broadcast_to(even[:, 0:1], (NW, 4))
    f16_idx = jnp.concatenate(
        [dummy, even[:, 0:4], even[:, 124:128], dummy], axis=1)

    pv32 = jnp.concatenate([prompt_embed[0:16], prompt_embed[4:20]], axis=0)

    out2 = _embed_call(idx_chunks, f16_idx, pv32, embed_table, B, D, new_len)
    out = out2.reshape(B, new_len, D)

    am3 = attention_mask.reshape(B, NB, 1, _K)
    lab3 = labels.reshape(B, NB, 1, _K)
    mask3, lab_out3 = _mask_labels_call(am3, lab3, B, NB)
    return out, mask3.reshape(B, new_len), lab_out3.reshape(B, new_len)
